# baseline (speedup 1.0000x reference)
import functools

import jax
import jax.numpy as jnp
from jax import lax
from jax.experimental import pallas as pl
from jax.experimental.pallas import tpu as pltpu
from jax.experimental.pallas import tpu_sc as plsc

N = 10000
E = 320000
D = 128
DE = 16

_NC = 2
_NS = 16
_NW = _NC * _NS
_BATCH = 128
_NBAT = E // _BATCH
_ROWS_N = N // _NS

_CHUNKS = 2
_CBAT = _NBAT // _CHUNKS
_CE = E // _CHUNKS


def _silu(x):
    return x * jax.nn.sigmoid(x)


def _ln(x, g, b, eps=1e-5):
    mu = jnp.mean(x, axis=-1, keepdims=True)
    var = jnp.var(x, axis=-1, keepdims=True)
    return (x - mu) / jnp.sqrt(var + eps) * g + b


def _worker_batches(pw, rem, wid, w_lt_rem):
    del w_lt_rem
    return wid * pw, pw


def _sc_gather_body(chunk, nb, p_hbm, src_hbm, g_hbm,
                    idx_v, rows_a, rows_b, gsem, wsem):
    cid = lax.axis_index("c")
    sid = lax.axis_index("s")
    wid = sid * _NC + cid
    pw = _CBAT // _NW
    rem = _CBAT - pw * _NW
    base = chunk * _CBAT
    start = base + wid * pw

    pltpu.sync_copy(src_hbm.at[pl.ds(start, pw)], idx_v)
    ngrp = pw // nb
    bufs = (rows_a, rows_b)
    wouts = [None] * ngrp
    for g in range(ngrp):
        buf = bufs[g % 2]
        if g >= 2:
            wouts[g - 2].wait()
        descs = [
            pltpu.async_copy(
                p_hbm.at[idx_v.at[g * nb + j]],
                buf.at[pl.ds(j * _BATCH, _BATCH)],
                gsem,
            )
            for j in range(nb)
        ]
        for d in descs:
            d.wait()
        wouts[g] = pltpu.async_copy(
            buf,
            g_hbm.at[pl.ds((wid * pw + g * nb) * _BATCH, nb * _BATCH)],
            wsem,
        )
    for g in range(max(ngrp - 2, 0), ngrp):
        wouts[g].wait()

    if rem:
        @pl.when(wid < rem)
        def _():
            b0 = pw * _NW + wid
            pltpu.sync_copy(src_hbm.at[pl.ds(base + b0, 1)],
                            idx_v.at[pl.ds(0, 1)])
            d = pltpu.async_copy(
                p_hbm.at[idx_v.at[0]], rows_a.at[pl.ds(0, _BATCH)], gsem)
            d.wait()
            pltpu.sync_copy(rows_a.at[pl.ds(0, _BATCH)],
                            g_hbm.at[pl.ds(b0 * _BATCH, _BATCH)])


def _sc_scatter_body(chunk, m_hbm, dst_hbm, zero_hbm, out_hbm,
                     idx_v, rows_a, rows_b, acc_sh, lsem, asem):
    cid = lax.axis_index("c")
    sid = lax.axis_index("s")
    wid = sid * _NC + cid
    pw = _CBAT // _NW
    rem = _CBAT - pw * _NW
    base = chunk * _CBAT

    pltpu.sync_copy(
        zero_hbm.at[pl.ds(sid * _ROWS_N, _ROWS_N)],
        acc_sh.at[pl.ds(sid * _ROWS_N, _ROWS_N)],
    )
    plsc.subcore_barrier()

    def do_pair(b0):
        pltpu.sync_copy(dst_hbm.at[pl.ds(b0, 2)], idx_v)
        la = pltpu.async_copy(
            m_hbm.at[pl.ds((b0 - base) * _BATCH, _BATCH)], rows_a, lsem)
        lb = pltpu.async_copy(
            m_hbm.at[pl.ds((b0 - base + 1) * _BATCH, _BATCH)], rows_b, lsem)
        la.wait()
        aa = pltpu.async_copy(rows_a, acc_sh.at[idx_v.at[0]], asem, add=True)
        lb.wait()
        ab = pltpu.async_copy(rows_b, acc_sh.at[idx_v.at[1]], asem, add=True)
        aa.wait()
        ab.wait()

    def do_one(b0):
        pltpu.sync_copy(dst_hbm.at[pl.ds(b0, 1)], idx_v.at[pl.ds(0, 1)])
        pltpu.sync_copy(
            m_hbm.at[pl.ds((b0 - base) * _BATCH, _BATCH)], rows_a)
        pltpu.sync_copy(rows_a, acc_sh.at[idx_v.at[0]], add=True)

    def body(i, carry):
        do_pair(base + wid * pw + i * 2)
        return carry

    lax.fori_loop(0, pw // 2, body, 0)

    if pw % 2:
        do_one(base + wid * pw + (pw // 2) * 2)

    if rem:
        @pl.when(wid < rem)
        def _():
            do_one(base + pw * _NW + wid)

    plsc.subcore_barrier()
    pltpu.sync_copy(
        acc_sh.at[pl.ds(sid * _ROWS_N, _ROWS_N)],
        out_hbm.at[cid, pl.ds(sid * _ROWS_N, _ROWS_N)],
    )


@functools.lru_cache(maxsize=None)
def _sc_kernels(chunk):
    mesh = plsc.VectorSubcoreMesh(
        core_axis_name="c", subcore_axis_name="s",
        num_cores=_NC, num_subcores=_NS)
    params = pltpu.CompilerParams(use_tc_tiling_on_sc=False)
    nb_g = 3
    gather = pl.kernel(
        functools.partial(_sc_gather_body, chunk, nb_g),
        out_type=jax.ShapeDtypeStruct((_CE, D), jnp.float32),
        mesh=mesh,
        compiler_params=params,
        scratch_types=[
            pltpu.VMEM((_CBAT // _NW, _BATCH), jnp.int32),
            pltpu.VMEM((nb_g * _BATCH, D), jnp.float32),
            pltpu.VMEM((nb_g * _BATCH, D), jnp.float32),
            pltpu.SemaphoreType.DMA,
            pltpu.SemaphoreType.DMA,
        ],
    )
    scatter = pl.kernel(
        functools.partial(_sc_scatter_body, chunk),
        out_type=jax.ShapeDtypeStruct((_NC, N, D), jnp.float32),
        mesh=mesh,
        compiler_params=params,
        scratch_types=[
            pltpu.VMEM((2, _BATCH), jnp.int32),
            pltpu.VMEM((_BATCH, D), jnp.float32),
            pltpu.VMEM((_BATCH, D), jnp.float32),
            pltpu.VMEM_SHARED((N, D), jnp.float32),
            pltpu.SemaphoreType.DMA,
            pltpu.SemaphoreType.DMA,
        ],
    )
    return gather, scatter


def _p_body(x_ref, w1a_ref, p_ref, z_ref):
    p_ref[...] = jnp.dot(x_ref[...], w1a_ref[...],
                         preferred_element_type=jnp.float32)
    z_ref[...] = jnp.zeros_like(z_ref)


_BE = 3200


def _mlp_body(g_ref, efp_ref, w1bd_ref, b1_ref, w2_ref, b2_ref, o_ref):
    qp = jnp.dot(efp_ref[...].astype(jnp.bfloat16), w1bd_ref[...],
                 preferred_element_type=jnp.float32)
    q = qp.reshape(_BE, D)
    m1 = _silu(g_ref[...] + (q + b1_ref[...]))
    o_ref[...] = _silu(jnp.dot(m1.astype(jnp.bfloat16), w2_ref[...],
                               preferred_element_type=jnp.float32) + b2_ref[...])


def _final_body(x_ref, pa_ref, pb_ref, w3_ref, b3_ref, w4_ref, b4_ref,
                g1_ref, be1_ref, g2_ref, be2_ref, o_ref):
    agg = (pa_ref[0] + pa_ref[1] + pb_ref[0] + pb_ref[1]) * 0.1
    h = _ln(x_ref[...] + agg, g1_ref[...], be1_ref[...])
    r = _silu(jnp.dot(h, w3_ref[...],
                      preferred_element_type=jnp.float32) + b3_ref[...])
    r = _silu(jnp.dot(r, w4_ref[...],
                      preferred_element_type=jnp.float32) + b4_ref[...])
    o_ref[...] = _ln(h + r, g2_ref[...], be2_ref[...])


def kernel(scalar_feats, edge_feats, edge_index, W1, b1, W2, b2, W3, b3,
           W4, b4, g1, be1, g2, be2):
    src2d = edge_index[0].reshape(_NBAT, _BATCH)
    dst2d = edge_index[1].reshape(_NBAT, _BATCH)
    efp = edge_feats.reshape(E // 8, 8 * DE)
    w1a = W1[:D]
    w1bd = jnp.kron(jnp.eye(8, dtype=W1.dtype), W1[D:]).astype(jnp.bfloat16)
    w2b = W2.astype(jnp.bfloat16)

    p, zeros = pl.pallas_call(
        _p_body,
        out_shape=[jax.ShapeDtypeStruct((N, D), jnp.float32),
                   jax.ShapeDtypeStruct((N, D), jnp.float32)],
    )(scalar_feats, w1a)
    steps = _CE // _BE
    parts = []
    for c in range(_CHUNKS):
        sc_gather, sc_scatter = _sc_kernels(c)
        g = sc_gather(p, src2d)
        m = pl.pallas_call(
            _mlp_body,
            grid=(steps,),
            in_specs=[
                pl.BlockSpec((_BE, D), lambda i: (i, 0)),
                pl.BlockSpec((_BE // 8, 8 * DE), lambda i, c=c: (i + c * steps, 0)),
                pl.BlockSpec((8 * DE, 8 * D), lambda i: (0, 0)),
                pl.BlockSpec((1, D), lambda i: (0, 0)),
                pl.BlockSpec((D, D), lambda i: (0, 0)),
                pl.BlockSpec((1, D), lambda i: (0, 0)),
            ],
            out_specs=pl.BlockSpec((_BE, D), lambda i: (i, 0)),
            out_shape=jax.ShapeDtypeStruct((_CE, D), jnp.float32),
        )(g, efp, w1bd, b1.reshape(1, D), w2b, b2.reshape(1, D))
        parts.append(sc_scatter(m, dst2d, zeros))

    out = pl.pallas_call(
        _final_body,
        out_shape=jax.ShapeDtypeStruct((N, D), jnp.float32),
    )(scalar_feats, parts[0], parts[1], W3, b3.reshape(1, D),
      W4, b4.reshape(1, D),
      g1.reshape(1, D), be1.reshape(1, D), g2.reshape(1, D), be2.reshape(1, D))
    return out


# speedup vs baseline: 1.0219x; 1.0219x over previous
import functools

import jax
import jax.numpy as jnp
from jax import lax
from jax.experimental import pallas as pl
from jax.experimental.pallas import tpu as pltpu
from jax.experimental.pallas import tpu_sc as plsc

N = 10000
E = 320000
D = 128
DE = 16

_NC = 2
_NS = 16
_NW = _NC * _NS
_BATCH = 128
_NBAT = E // _BATCH
_ROWS_N = N // _NS

_CHUNKS = 2
_CBAT = _NBAT // _CHUNKS
_CE = E // _CHUNKS


def _silu(x):
    return x * jax.nn.sigmoid(x)


def _ln(x, g, b, eps=1e-5):
    mu = jnp.mean(x, axis=-1, keepdims=True)
    var = jnp.var(x, axis=-1, keepdims=True)
    return (x - mu) / jnp.sqrt(var + eps) * g + b


def _worker_batches(pw, rem, wid, w_lt_rem):
    del w_lt_rem
    return wid * pw, pw


def _sc_gather_body(chunk, nb, p_hbm, src_hbm, g_hbm, idx_v, rows_v, sem):
    cid = lax.axis_index("c")
    sid = lax.axis_index("s")
    wid = sid * _NC + cid
    pw = _CBAT // _NW
    rem = _CBAT - pw * _NW
    base = chunk * _CBAT

    def do_chunk(b0, nbatch):
        pltpu.sync_copy(src_hbm.at[pl.ds(b0, nbatch)], idx_v.at[pl.ds(0, nbatch)])
        descs = [
            pltpu.async_copy(
                p_hbm.at[idx_v.at[j]],
                rows_v.at[pl.ds(j * _BATCH, _BATCH)],
                sem,
            )
            for j in range(nbatch)
        ]
        for d in descs:
            d.wait()
        pltpu.sync_copy(
            rows_v.at[pl.ds(0, nbatch * _BATCH)],
            g_hbm.at[pl.ds((b0 - base) * _BATCH, nbatch * _BATCH)],
        )

    def body(i, carry):
        do_chunk(base + wid * pw + i * nb, nb)
        return carry

    lax.fori_loop(0, pw // nb, body, 0)

    if rem:
        @pl.when(wid < rem)
        def _():
            do_chunk(base + pw * _NW + wid, 1)


def _sc_scatter_body(chunk, m_hbm, dst_hbm, zero_hbm, out_hbm,
                     idx_v, rows_a, rows_b, acc_sh, lsem, asem):
    cid = lax.axis_index("c")
    sid = lax.axis_index("s")
    wid = sid * _NC + cid
    pw = _CBAT // _NW
    rem = _CBAT - pw * _NW
    base = chunk * _CBAT

    pltpu.sync_copy(
        zero_hbm.at[pl.ds(sid * _ROWS_N, _ROWS_N)],
        acc_sh.at[pl.ds(sid * _ROWS_N, _ROWS_N)],
    )
    plsc.subcore_barrier()

    def do_pair(b0):
        pltpu.sync_copy(dst_hbm.at[pl.ds(b0, 2)], idx_v)
        la = pltpu.async_copy(
            m_hbm.at[pl.ds((b0 - base) * _BATCH, _BATCH)], rows_a, lsem)
        lb = pltpu.async_copy(
            m_hbm.at[pl.ds((b0 - base + 1) * _BATCH, _BATCH)], rows_b, lsem)
        la.wait()
        aa = pltpu.async_copy(rows_a, acc_sh.at[idx_v.at[0]], asem, add=True)
        lb.wait()
        ab = pltpu.async_copy(rows_b, acc_sh.at[idx_v.at[1]], asem, add=True)
        aa.wait()
        ab.wait()

    def do_one(b0):
        pltpu.sync_copy(dst_hbm.at[pl.ds(b0, 1)], idx_v.at[pl.ds(0, 1)])
        pltpu.sync_copy(
            m_hbm.at[pl.ds((b0 - base) * _BATCH, _BATCH)], rows_a)
        pltpu.sync_copy(rows_a, acc_sh.at[idx_v.at[0]], add=True)

    def body(i, carry):
        do_pair(base + wid * pw + i * 2)
        return carry

    lax.fori_loop(0, pw // 2, body, 0)

    if pw % 2:
        do_one(base + wid * pw + (pw // 2) * 2)

    if rem:
        @pl.when(wid < rem)
        def _():
            do_one(base + pw * _NW + wid)

    plsc.subcore_barrier()
    pltpu.sync_copy(
        acc_sh.at[pl.ds(sid * _ROWS_N, _ROWS_N)],
        out_hbm.at[cid, pl.ds(sid * _ROWS_N, _ROWS_N)],
    )


@functools.lru_cache(maxsize=None)
def _sc_kernels(chunk):
    mesh = plsc.VectorSubcoreMesh(
        core_axis_name="c", subcore_axis_name="s",
        num_cores=_NC, num_subcores=_NS)
    params = pltpu.CompilerParams(use_tc_tiling_on_sc=False)
    nb_g = 3
    gather = pl.kernel(
        functools.partial(_sc_gather_body, chunk, nb_g),
        out_type=jax.ShapeDtypeStruct((_CE, D), jnp.float32),
        mesh=mesh,
        compiler_params=params,
        scratch_types=[
            pltpu.VMEM((nb_g, _BATCH), jnp.int32),
            pltpu.VMEM((nb_g * _BATCH, D), jnp.float32),
            pltpu.SemaphoreType.DMA,
        ],
    )
    scatter = pl.kernel(
        functools.partial(_sc_scatter_body, chunk),
        out_type=jax.ShapeDtypeStruct((_NC, N, D), jnp.float32),
        mesh=mesh,
        compiler_params=params,
        scratch_types=[
            pltpu.VMEM((2, _BATCH), jnp.int32),
            pltpu.VMEM((_BATCH, D), jnp.float32),
            pltpu.VMEM((_BATCH, D), jnp.float32),
            pltpu.VMEM_SHARED((N, D), jnp.float32),
            pltpu.SemaphoreType.DMA,
            pltpu.SemaphoreType.DMA,
        ],
    )
    return gather, scatter


def _p_body(x_ref, w1a_ref, p_ref, z_ref):
    p_ref[...] = jnp.dot(x_ref[...], w1a_ref[...],
                         preferred_element_type=jnp.float32)
    z_ref[...] = jnp.zeros_like(z_ref)


_BE = 3200


def _mlp_body(g_ref, efp_ref, w1bd_ref, b1_ref, w2_ref, b2_ref, o_ref):
    qp = jnp.dot(efp_ref[...].astype(jnp.bfloat16), w1bd_ref[...],
                 preferred_element_type=jnp.float32)
    q = qp.reshape(_BE, D)
    m1 = _silu(g_ref[...] + (q + b1_ref[...]))
    o_ref[...] = _silu(jnp.dot(m1.astype(jnp.bfloat16), w2_ref[...],
                               preferred_element_type=jnp.float32) + b2_ref[...])


def _final_body(x_ref, pa_ref, pb_ref, w3_ref, b3_ref, w4_ref, b4_ref,
                g1_ref, be1_ref, g2_ref, be2_ref, o_ref):
    agg = (pa_ref[0] + pa_ref[1] + pb_ref[0] + pb_ref[1]) * 0.1
    h = _ln(x_ref[...] + agg, g1_ref[...], be1_ref[...])
    r = _silu(jnp.dot(h, w3_ref[...],
                      preferred_element_type=jnp.float32) + b3_ref[...])
    r = _silu(jnp.dot(r, w4_ref[...],
                      preferred_element_type=jnp.float32) + b4_ref[...])
    o_ref[...] = _ln(h + r, g2_ref[...], be2_ref[...])


def kernel(scalar_feats, edge_feats, edge_index, W1, b1, W2, b2, W3, b3,
           W4, b4, g1, be1, g2, be2):
    src2d = edge_index[0].reshape(_NBAT, _BATCH)
    dst2d = edge_index[1].reshape(_NBAT, _BATCH)
    efp = edge_feats.reshape(E // 8, 8 * DE)
    w1a = W1[:D]
    w1bd = jnp.kron(jnp.eye(8, dtype=W1.dtype), W1[D:]).astype(jnp.bfloat16)
    w2b = W2.astype(jnp.bfloat16)

    p, zeros = pl.pallas_call(
        _p_body,
        out_shape=[jax.ShapeDtypeStruct((N, D), jnp.float32),
                   jax.ShapeDtypeStruct((N, D), jnp.float32)],
    )(scalar_feats, w1a)
    steps = _CE // _BE
    parts = []
    for c in range(_CHUNKS):
        sc_gather, sc_scatter = _sc_kernels(c)
        g = sc_gather(p, src2d)
        m = pl.pallas_call(
            _mlp_body,
            grid=(steps,),
            in_specs=[
                pl.BlockSpec((_BE, D), lambda i: (i, 0)),
                pl.BlockSpec((_BE // 8, 8 * DE), lambda i, c=c: (i + c * steps, 0)),
                pl.BlockSpec((8 * DE, 8 * D), lambda i: (0, 0)),
                pl.BlockSpec((1, D), lambda i: (0, 0)),
                pl.BlockSpec((D, D), lambda i: (0, 0)),
                pl.BlockSpec((1, D), lambda i: (0, 0)),
            ],
            out_specs=pl.BlockSpec((_BE, D), lambda i: (i, 0)),
            out_shape=jax.ShapeDtypeStruct((_CE, D), jnp.float32),
        )(g, efp, w1bd, b1.reshape(1, D), w2b, b2.reshape(1, D))
        parts.append(sc_scatter(m, dst2d, zeros))

    out = pl.pallas_call(
        _final_body,
        out_shape=jax.ShapeDtypeStruct((N, D), jnp.float32),
    )(scalar_feats, parts[0], parts[1], W3, b3.reshape(1, D),
      W4, b4.reshape(1, D),
      g1.reshape(1, D), be1.reshape(1, D), g2.reshape(1, D), be2.reshape(1, D))
    return out


# speedup vs baseline: 1.0271x; 1.0052x over previous
import functools

import jax
import jax.numpy as jnp
from jax import lax
from jax.experimental import pallas as pl
from jax.experimental.pallas import tpu as pltpu
from jax.experimental.pallas import tpu_sc as plsc

N = 10000
E = 320000
D = 128
DE = 16

_NC = 2
_NS = 16
_NW = _NC * _NS
_BATCH = 128
_NBAT = E // _BATCH
_ROWS_N = N // _NS

_CHUNKS = 2
_CBAT = _NBAT // _CHUNKS
_CE = E // _CHUNKS


def _silu(x):
    return x * jax.nn.sigmoid(x)


def _ln(x, g, b, eps=1e-5):
    mu = jnp.mean(x, axis=-1, keepdims=True)
    var = jnp.var(x, axis=-1, keepdims=True)
    return (x - mu) / jnp.sqrt(var + eps) * g + b


def _worker_batches(pw, rem, wid, w_lt_rem):
    del w_lt_rem
    return wid * pw, pw


def _sc_gather_body(chunk, nb, p_hbm, src_hbm, g_hbm, idx_v, rows_v, sem):
    cid = lax.axis_index("c")
    sid = lax.axis_index("s")
    wid = sid * _NC + cid
    pw = _CBAT // _NW
    rem = _CBAT - pw * _NW
    base = chunk * _CBAT

    def do_chunk(b0, nbatch):
        pltpu.sync_copy(src_hbm.at[pl.ds(b0, nbatch)], idx_v.at[pl.ds(0, nbatch)])
        descs = [
            pltpu.async_copy(
                p_hbm.at[idx_v.at[j]],
                rows_v.at[pl.ds(j * _BATCH, _BATCH)],
                sem,
            )
            for j in range(nbatch)
        ]
        for d in descs:
            d.wait()
        pltpu.sync_copy(
            rows_v.at[pl.ds(0, nbatch * _BATCH)],
            g_hbm.at[pl.ds((b0 - base) * _BATCH, nbatch * _BATCH)],
        )

    def body(i, carry):
        do_chunk(base + wid * pw + i * nb, nb)
        return carry

    lax.fori_loop(0, pw // nb, body, 0)

    if rem:
        @pl.when(wid < rem)
        def _():
            do_chunk(base + pw * _NW + wid, 1)


def _sc_scatter_body(chunk, m_hbm, dst_hbm, zero_hbm, out_hbm,
                     idx_v, rows_a, rows_b, acc_sh, lsem, asem):
    cid = lax.axis_index("c")
    sid = lax.axis_index("s")
    wid = sid * _NC + cid
    pw = _CBAT // _NW
    rem = _CBAT - pw * _NW
    base = chunk * _CBAT

    pltpu.sync_copy(
        zero_hbm.at[pl.ds(sid * _ROWS_N, _ROWS_N)],
        acc_sh.at[pl.ds(sid * _ROWS_N, _ROWS_N)],
    )
    plsc.subcore_barrier()

    def do_pair(b0):
        pltpu.sync_copy(dst_hbm.at[pl.ds(b0, 2)], idx_v)
        la = pltpu.async_copy(
            m_hbm.at[pl.ds((b0 - base) * _BATCH, _BATCH)], rows_a, lsem)
        lb = pltpu.async_copy(
            m_hbm.at[pl.ds((b0 - base + 1) * _BATCH, _BATCH)], rows_b, lsem)
        la.wait()
        aa = pltpu.async_copy(rows_a, acc_sh.at[idx_v.at[0]], asem, add=True)
        lb.wait()
        ab = pltpu.async_copy(rows_b, acc_sh.at[idx_v.at[1]], asem, add=True)
        aa.wait()
        ab.wait()

    def do_one(b0):
        pltpu.sync_copy(dst_hbm.at[pl.ds(b0, 1)], idx_v.at[pl.ds(0, 1)])
        pltpu.sync_copy(
            m_hbm.at[pl.ds((b0 - base) * _BATCH, _BATCH)], rows_a)
        pltpu.sync_copy(rows_a, acc_sh.at[idx_v.at[0]], add=True)

    def body(i, carry):
        do_pair(base + wid * pw + i * 2)
        return carry

    lax.fori_loop(0, pw // 2, body, 0)

    if pw % 2:
        do_one(base + wid * pw + (pw // 2) * 2)

    if rem:
        @pl.when(wid < rem)
        def _():
            do_one(base + pw * _NW + wid)

    plsc.subcore_barrier()
    pltpu.sync_copy(
        acc_sh.at[pl.ds(sid * _ROWS_N, _ROWS_N)],
        out_hbm.at[cid, pl.ds(sid * _ROWS_N, _ROWS_N)],
    )


@functools.lru_cache(maxsize=None)
def _sc_kernels(chunk):
    mesh = plsc.VectorSubcoreMesh(
        core_axis_name="c", subcore_axis_name="s",
        num_cores=_NC, num_subcores=_NS)
    params = pltpu.CompilerParams(use_tc_tiling_on_sc=False)
    nb_g = 3
    gather = pl.kernel(
        functools.partial(_sc_gather_body, chunk, nb_g),
        out_type=jax.ShapeDtypeStruct((_CE, D), jnp.float32),
        mesh=mesh,
        compiler_params=params,
        scratch_types=[
            pltpu.VMEM((nb_g, _BATCH), jnp.int32),
            pltpu.VMEM((nb_g * _BATCH, D), jnp.float32),
            pltpu.SemaphoreType.DMA,
        ],
    )
    scatter = pl.kernel(
        functools.partial(_sc_scatter_body, chunk),
        out_type=jax.ShapeDtypeStruct((_NC, N, D), jnp.float32),
        mesh=mesh,
        compiler_params=params,
        scratch_types=[
            pltpu.VMEM((2, _BATCH), jnp.int32),
            pltpu.VMEM((_BATCH, D), jnp.float32),
            pltpu.VMEM((_BATCH, D), jnp.float32),
            pltpu.VMEM_SHARED((N, D), jnp.float32),
            pltpu.SemaphoreType.DMA,
            pltpu.SemaphoreType.DMA,
        ],
    )
    return gather, scatter


_BE = 3200


def _mlp_body(g_ref, efp_ref, w1a_ref, w1bd_ref, b1_ref, w2_ref, b2_ref,
              o_ref):
    pg = jnp.dot(g_ref[...].astype(jnp.bfloat16), w1a_ref[...],
                 preferred_element_type=jnp.float32)
    qp = jnp.dot(efp_ref[...].astype(jnp.bfloat16), w1bd_ref[...],
                 preferred_element_type=jnp.float32)
    q = qp.reshape(_BE, D)
    m1 = _silu(pg + (q + b1_ref[...]))
    o_ref[...] = _silu(jnp.dot(m1.astype(jnp.bfloat16), w2_ref[...],
                               preferred_element_type=jnp.float32) + b2_ref[...])


def _final_body(x_ref, pa_ref, pb_ref, w3_ref, b3_ref, w4_ref, b4_ref,
                g1_ref, be1_ref, g2_ref, be2_ref, o_ref):
    agg = (pa_ref[0] + pa_ref[1] + pb_ref[0] + pb_ref[1]) * 0.1
    h = _ln(x_ref[...] + agg, g1_ref[...], be1_ref[...])
    r = _silu(jnp.dot(h, w3_ref[...],
                      preferred_element_type=jnp.float32) + b3_ref[...])
    r = _silu(jnp.dot(r, w4_ref[...],
                      preferred_element_type=jnp.float32) + b4_ref[...])
    o_ref[...] = _ln(h + r, g2_ref[...], be2_ref[...])


def kernel(scalar_feats, edge_feats, edge_index, W1, b1, W2, b2, W3, b3,
           W4, b4, g1, be1, g2, be2):
    src2d = edge_index[0].reshape(_NBAT, _BATCH)
    dst2d = edge_index[1].reshape(_NBAT, _BATCH)
    efp = edge_feats.reshape(E // 8, 8 * DE)
    w1ab = W1[:D].astype(jnp.bfloat16)
    w1bd = jnp.kron(jnp.eye(8, dtype=W1.dtype), W1[D:]).astype(jnp.bfloat16)
    w2b = W2.astype(jnp.bfloat16)

    zeros = jnp.zeros((N, D), jnp.float32)
    steps = _CE // _BE
    parts = []
    for c in range(_CHUNKS):
        sc_gather, sc_scatter = _sc_kernels(c)
        g = sc_gather(scalar_feats, src2d)
        m = pl.pallas_call(
            _mlp_body,
            grid=(steps,),
            in_specs=[
                pl.BlockSpec((_BE, D), lambda i: (i, 0)),
                pl.BlockSpec((_BE // 8, 8 * DE), lambda i, c=c: (i + c * steps, 0)),
                pl.BlockSpec((D, D), lambda i: (0, 0)),
                pl.BlockSpec((8 * DE, 8 * D), lambda i: (0, 0)),
                pl.BlockSpec((1, D), lambda i: (0, 0)),
                pl.BlockSpec((D, D), lambda i: (0, 0)),
                pl.BlockSpec((1, D), lambda i: (0, 0)),
            ],
            out_specs=pl.BlockSpec((_BE, D), lambda i: (i, 0)),
            out_shape=jax.ShapeDtypeStruct((_CE, D), jnp.float32),
        )(g, efp, w1ab, w1bd, b1.reshape(1, D), w2b, b2.reshape(1, D))
        parts.append(sc_scatter(m, dst2d, zeros))

    out = pl.pallas_call(
        _final_body,
        out_shape=jax.ShapeDtypeStruct((N, D), jnp.float32),
    )(scalar_feats, parts[0], parts[1], W3, b3.reshape(1, D),
      W4, b4.reshape(1, D),
      g1.reshape(1, D), be1.reshape(1, D), g2.reshape(1, D), be2.reshape(1, D))
    return out
